# baseline (speedup 1.0000x reference)
import functools

import jax
import jax.numpy as jnp
from jax import lax
from jax.experimental import pallas as pl
from jax.experimental.pallas import tpu as pltpu
from jax.experimental.pallas import tpu_sc as plsc

NC = 2
NS = 16
LANES = 16


def _xm_body(x_ref, w_ref, o_ref):
    o_ref[...] = jnp.dot(x_ref[...], w_ref[...], preferred_element_type=jnp.float32)


def _r_body(a_ref, w_ref, b_ref, o_ref):
    o_ref[...] = (
        jnp.dot(a_ref[...], w_ref[...], preferred_element_type=jnp.float32)
        + b_ref[...]
    )


def _tail_body(x_ref, a0_ref, a1_ref, wself_ref, wsi_ref, bsi_ref, wnl_ref,
               bnl_ref, wfin_ref, bfin_ref, o_ref):
    h = (
        jnp.dot(x_ref[...], wself_ref[...], preferred_element_type=jnp.float32)
        + a0_ref[...]
        + a1_ref[...]
    )
    h = jnp.dot(h, wsi_ref[...], preferred_element_type=jnp.float32) + bsi_ref[...]
    h = jnp.maximum(
        jnp.dot(h, wnl_ref[...], preferred_element_type=jnp.float32) + bnl_ref[...],
        0.0,
    )
    o_ref[...] = (
        jnp.dot(h, wfin_ref[...], preferred_element_type=jnp.float32) + bfin_ref[...]
    )


def _make_sc_scatter(N, E, D, K):
    E_core = E // NC
    E_tile = E_core // NS
    n_chunks = E_tile // K
    rem = E_tile - n_chunks * K
    assert rem == 0 or (rem % 8 == 0 and rem <= 128)
    NR = -(-(N // NS) // 8) * 8
    NR_LAST = N - (NS - 1) * NR
    assert NR_LAST > 0 and NR_LAST % 8 == 0
    mesh = plsc.VectorSubcoreMesh(core_axis_name="c", subcore_axis_name="s")

    assert n_chunks >= 3

    @functools.partial(
        pl.kernel,
        mesh=mesh,
        out_type=jax.ShapeDtypeStruct((NC * N, D), jnp.float32),
        scratch_types=[
            pltpu.VMEM((2, K), jnp.int32),
            pltpu.VMEM((2, K), jnp.int32),
            pltpu.VMEM((K,), jnp.int32),
            pltpu.VMEM((K,), jnp.int32),
            pltpu.VMEM((max(rem, 8),), jnp.int32),
            pltpu.VMEM((max(rem, 8),), jnp.int32),
            pltpu.VMEM((2, K, D), jnp.float32),
            pltpu.VMEM((2, K, D), jnp.float32),
            pltpu.VMEM((2, K, D), jnp.float32),
            pltpu.VMEM_SHARED((N, D), jnp.float32),
            pltpu.SemaphoreType.DMA,
            pltpu.SemaphoreType.DMA,
            pltpu.SemaphoreType.DMA,
            pltpu.SemaphoreType.DMA,
            pltpu.SemaphoreType.DMA,
            pltpu.SemaphoreType.DMA,
        ],
    )
    def sc_scatter(xm_hbm, r_hbm, src_hbm, dst_hbm, zeros_hbm, out_hbm,
                   src_v, dst_v, dst_sc0, dst_sc1, srcr_v, dstr_v, rows_v,
                   rv_v, msg_v, agg_s, sem_in0, sem_in1, sem_g0, sem_g1,
                   sem_s0, sem_s1):
        dst_sc = (dst_sc0, dst_sc1)
        c = lax.axis_index("c")
        s = lax.axis_index("s")

        @pl.when(s < NS - 1)
        def _():
            pltpu.sync_copy(zeros_hbm.at[pl.ds(s * NR, NR)],
                            agg_s.at[pl.ds(s * NR, NR)])

        @pl.when(s == NS - 1)
        def _():
            pltpu.sync_copy(zeros_hbm.at[pl.ds((NS - 1) * NR, NR_LAST)],
                            agg_s.at[pl.ds((NS - 1) * NR, NR_LAST)])

        plsc.subcore_barrier()

        base0 = c * E_core + s * E_tile
        sem_in = (sem_in0, sem_in1)
        sem_g = (sem_g0, sem_g1)
        sem_s = (sem_s0, sem_s1)

        def a_start(i, b):
            base = base0 + i * K
            pltpu.async_copy(src_hbm.at[pl.ds(base, K)], src_v.at[b], sem_in[b])
            pltpu.async_copy(dst_hbm.at[pl.ds(base, K)], dst_v.at[b], sem_in[b])
            pltpu.async_copy(r_hbm.at[pl.ds(base, K)], rv_v.at[b], sem_in[b])

        def a_wait(i, b):
            base = base0 + i * K
            pltpu.make_async_copy(src_hbm.at[pl.ds(base, K)], src_v.at[b],
                                  sem_in[b]).wait()
            pltpu.make_async_copy(dst_hbm.at[pl.ds(base, K)], dst_v.at[b],
                                  sem_in[b]).wait()
            pltpu.make_async_copy(r_hbm.at[pl.ds(base, K)], rv_v.at[b],
                                  sem_in[b]).wait()

        def g_start(b):
            pltpu.async_copy(xm_hbm.at[src_v.at[b]], rows_v.at[b], sem_g[b])

        def g_wait(b):
            pltpu.make_async_copy(xm_hbm.at[src_v.at[b]], rows_v.at[b],
                                  sem_g[b]).wait()

        def scatter_start(b):
            pltpu.async_copy(msg_v.at[b], agg_s.at[dst_sc[b]], sem_s[b],
                             add=True)

        def scatter_wait(b):
            pltpu.make_async_copy(msg_v.at[b], agg_s.at[dst_sc[b]],
                                  sem_s[b]).wait()

        def mul_rows(b, nrows):
            def mul_row(k, cc):
                prods = []
                for j in range(D // LANES):
                    sl = pl.ds(j * LANES, LANES)
                    prods.append(rows_v[b, k, sl] * rv_v[b, k, sl])
                for j in range(D // LANES):
                    sl = pl.ds(j * LANES, LANES)
                    msg_v[b, k, sl] = prods[j]
                return cc

            lax.fori_loop(0, nrows, mul_row, 0, unroll=8)

        def mul_scatter(b, wait_pred):
            if wait_pred is True:
                scatter_wait(b)
            elif wait_pred is not False:
                @pl.when(wait_pred)
                def _():
                    scatter_wait(b)
            offs = list(range(0, K - LANES + 1, LANES))
            if K % LANES:
                offs.append(K - LANES)
            for o in offs:
                sl = pl.ds(o, LANES)
                dst_sc[b][sl] = dst_v[b, sl]
            mul_rows(b, K)
            scatter_start(b)

        a_start(0, 0)
        a_wait(0, 0)
        g_start(0)
        a_start(1, 1)

        def pair_body(t, carry):
            g = 2 * t
            warm = t >= 1
            a_wait(g + 1, 1)
            g_start(1)
            g_wait(0)
            mul_scatter(0, warm)
            a_start(g + 2, 0)
            a_wait(g + 2, 0)
            g_start(0)
            g_wait(1)
            mul_scatter(1, warm)

            @pl.when(g + 3 < n_chunks)
            def _():
                a_start(g + 3, 1)

            return carry

        lax.fori_loop(0, (n_chunks - 1) // 2, pair_body, 0)
        if n_chunks % 2 == 1:
            g_wait(0)
            mul_scatter(0, True)
        else:
            a_wait(n_chunks - 1, 1)
            g_start(1)
            g_wait(0)
            mul_scatter(0, True)
            g_wait(1)
            mul_scatter(1, True)
        scatter_wait(0)
        scatter_wait(1)

        if rem:
            base = base0 + n_chunks * K
            pltpu.sync_copy(src_hbm.at[pl.ds(base, rem)], srcr_v)
            pltpu.sync_copy(dst_hbm.at[pl.ds(base, rem)], dstr_v)
            pltpu.sync_copy(r_hbm.at[pl.ds(base, rem)],
                            rv_v.at[0, pl.ds(0, rem)])
            pltpu.async_copy(xm_hbm.at[srcr_v], rows_v.at[0, pl.ds(0, rem)],
                             sem_g0).wait()
            mul_rows(0, rem)
            pltpu.sync_copy(msg_v.at[0, pl.ds(0, rem)], agg_s.at[dstr_v],
                            add=True)

        plsc.subcore_barrier()

        @pl.when(s < NS - 1)
        def _():
            pltpu.sync_copy(agg_s.at[pl.ds(s * NR, NR)],
                            out_hbm.at[pl.ds(c * N + s * NR, NR)])

        @pl.when(s == NS - 1)
        def _():
            pltpu.sync_copy(agg_s.at[pl.ds((NS - 1) * NR, NR_LAST)],
                            out_hbm.at[pl.ds(c * N + (NS - 1) * NR, NR_LAST)])

    return sc_scatter


def kernel(x, edge_index, edge_attr, W_msg, W_self, W_r, b_r, W_si, b_si,
           W_nl, b_nl, W_fin, b_fin):
    N, D = x.shape
    E = edge_index.shape[1]
    DE = edge_attr.shape[1]
    H = W_si.shape[1]
    C = W_fin.shape[1]

    BN = 1000
    xm = pl.pallas_call(
        _xm_body,
        grid=(N // BN,),
        in_specs=[
            pl.BlockSpec((BN, D), lambda i: (i, 0)),
            pl.BlockSpec((D, D), lambda i: (0, 0)),
        ],
        out_specs=pl.BlockSpec((BN, D), lambda i: (i, 0)),
        out_shape=jax.ShapeDtypeStruct((N, D), jnp.float32),
    )(x, W_msg)

    BE = 2000
    r = pl.pallas_call(
        _r_body,
        grid=(E // BE,),
        in_specs=[
            pl.BlockSpec((BE, DE), lambda i: (i, 0)),
            pl.BlockSpec((DE, D), lambda i: (0, 0)),
            pl.BlockSpec((1, D), lambda i: (0, 0)),
        ],
        out_specs=pl.BlockSpec((BE, D), lambda i: (i, 0)),
        out_shape=jax.ShapeDtypeStruct((E, D), jnp.float32),
    )(edge_attr, W_r, b_r.reshape(1, D))

    K = 64
    src = edge_index[0]
    dst = edge_index[1]
    zeros = jnp.zeros((N, D), jnp.float32)
    parts = _make_sc_scatter(N, E, D, K)(xm, r, src, dst, zeros)

    out = pl.pallas_call(
        _tail_body,
        grid=(N // BN,),
        in_specs=[
            pl.BlockSpec((BN, D), lambda i: (i, 0)),
            pl.BlockSpec((BN, D), lambda i: (i, 0)),
            pl.BlockSpec((BN, D), lambda i: (i + N // BN, 0)),
            pl.BlockSpec((D, D), lambda i: (0, 0)),
            pl.BlockSpec((D, H), lambda i: (0, 0)),
            pl.BlockSpec((1, H), lambda i: (0, 0)),
            pl.BlockSpec((H, H), lambda i: (0, 0)),
            pl.BlockSpec((1, H), lambda i: (0, 0)),
            pl.BlockSpec((H, C), lambda i: (0, 0)),
            pl.BlockSpec((1, C), lambda i: (0, 0)),
        ],
        out_specs=pl.BlockSpec((BN, C), lambda i: (i, 0)),
        out_shape=jax.ShapeDtypeStruct((N, C), jnp.float32),
    )(x, parts, parts, W_self, W_si, b_si.reshape(1, H), W_nl,
      b_nl.reshape(1, H), W_fin, b_fin.reshape(1, C))

    return out


# speedup vs baseline: 1.1188x; 1.1188x over previous
import functools

import jax
import jax.numpy as jnp
from jax import lax
from jax.experimental import pallas as pl
from jax.experimental.pallas import tpu as pltpu
from jax.experimental.pallas import tpu_sc as plsc

NC = 2
NS = 16
LANES = 16


def _xm_body(x_ref, w_ref, o_ref):
    o_ref[...] = jnp.dot(x_ref[...], w_ref[...], preferred_element_type=jnp.float32)


def _r_body(a_ref, w_ref, b_ref, o_ref):
    o_ref[...] = (
        jnp.dot(a_ref[...], w_ref[...], preferred_element_type=jnp.float32)
        + b_ref[...]
    )


def _tail_body(x_ref, a0_ref, a1_ref, wself_ref, wsi_ref, bsi_ref, wnl_ref,
               bnl_ref, wfin_ref, bfin_ref, o_ref):
    h = (
        jnp.dot(x_ref[...], wself_ref[...], preferred_element_type=jnp.float32)
        + a0_ref[...]
        + a1_ref[...]
    )
    h = jnp.dot(h, wsi_ref[...], preferred_element_type=jnp.float32) + bsi_ref[...]
    h = jnp.maximum(
        jnp.dot(h, wnl_ref[...], preferred_element_type=jnp.float32) + bnl_ref[...],
        0.0,
    )
    o_ref[...] = (
        jnp.dot(h, wfin_ref[...], preferred_element_type=jnp.float32) + bfin_ref[...]
    )


def _make_sc_scatter(N, E, D, K):
    E_core = E // NC
    E_tile = E_core // NS
    n_chunks = E_tile // K
    rem = E_tile - n_chunks * K
    assert rem == 0 or (rem % 8 == 0 and rem <= 128)
    NR = -(-(N // NS) // 8) * 8
    NR_LAST = N - (NS - 1) * NR
    assert NR_LAST > 0 and NR_LAST % 8 == 0
    mesh = plsc.VectorSubcoreMesh(core_axis_name="c", subcore_axis_name="s")

    assert n_chunks >= 3

    @functools.partial(
        pl.kernel,
        mesh=mesh,
        out_type=jax.ShapeDtypeStruct((NC * N, D), jnp.float32),
        scratch_types=[
            pltpu.VMEM((2, K), jnp.int32),
            pltpu.VMEM((2, K), jnp.int32),
            pltpu.VMEM((K,), jnp.int32),
            pltpu.VMEM((K,), jnp.int32),
            pltpu.VMEM((max(rem, 8),), jnp.int32),
            pltpu.VMEM((max(rem, 8),), jnp.int32),
            pltpu.VMEM((2, K, D), jnp.float32),
            pltpu.VMEM((2, K, D), jnp.float32),
            pltpu.VMEM((2, K, D), jnp.float32),
            pltpu.VMEM_SHARED((N, D), jnp.float32),
            pltpu.SemaphoreType.DMA,
            pltpu.SemaphoreType.DMA,
            pltpu.SemaphoreType.DMA,
            pltpu.SemaphoreType.DMA,
            pltpu.SemaphoreType.DMA,
            pltpu.SemaphoreType.DMA,
        ],
    )
    def sc_scatter(xm_hbm, r_hbm, src_hbm, dst_hbm, zeros_hbm, out_hbm,
                   src_v, dst_v, dst_sc0, dst_sc1, srcr_v, dstr_v, rows_v,
                   rv_v, msg_v, agg_s, sem_in0, sem_in1, sem_g0, sem_g1,
                   sem_s0, sem_s1):
        dst_sc = (dst_sc0, dst_sc1)
        c = lax.axis_index("c")
        s = lax.axis_index("s")

        @pl.when(s < NS - 1)
        def _():
            pltpu.sync_copy(zeros_hbm.at[pl.ds(s * NR, NR)],
                            agg_s.at[pl.ds(s * NR, NR)])

        @pl.when(s == NS - 1)
        def _():
            pltpu.sync_copy(zeros_hbm.at[pl.ds((NS - 1) * NR, NR_LAST)],
                            agg_s.at[pl.ds((NS - 1) * NR, NR_LAST)])

        plsc.subcore_barrier()

        base0 = c * E_core + s * E_tile
        sem_in = (sem_in0, sem_in1)
        sem_g = (sem_g0, sem_g1)
        sem_s = (sem_s0, sem_s1)

        def a_start(i, b):
            base = base0 + i * K
            pltpu.async_copy(src_hbm.at[pl.ds(base, K)], src_v.at[b], sem_in[b])
            pltpu.async_copy(dst_hbm.at[pl.ds(base, K)], dst_v.at[b], sem_in[b])
            pltpu.async_copy(r_hbm.at[pl.ds(base, K)], rv_v.at[b], sem_in[b])

        def a_wait(i, b):
            base = base0 + i * K
            pltpu.make_async_copy(src_hbm.at[pl.ds(base, K)], src_v.at[b],
                                  sem_in[b]).wait()
            pltpu.make_async_copy(dst_hbm.at[pl.ds(base, K)], dst_v.at[b],
                                  sem_in[b]).wait()
            pltpu.make_async_copy(r_hbm.at[pl.ds(base, K)], rv_v.at[b],
                                  sem_in[b]).wait()

        def g_start(b):
            pltpu.async_copy(xm_hbm.at[src_v.at[b]], rows_v.at[b], sem_g[b])

        def g_wait(b):
            pltpu.make_async_copy(xm_hbm.at[src_v.at[b]], rows_v.at[b],
                                  sem_g[b]).wait()

        def scatter_start(b):
            pltpu.async_copy(msg_v.at[b], agg_s.at[dst_sc[b]], sem_s[b],
                             add=True)

        def scatter_wait(b):
            pltpu.make_async_copy(msg_v.at[b], agg_s.at[dst_sc[b]],
                                  sem_s[b]).wait()

        def mul_rows(b, nrows):
            def mul_row(k, cc):
                prods = []
                for j in range(D // LANES):
                    sl = pl.ds(j * LANES, LANES)
                    prods.append(rows_v[b, k, sl] * rv_v[b, k, sl])
                for j in range(D // LANES):
                    sl = pl.ds(j * LANES, LANES)
                    msg_v[b, k, sl] = prods[j]
                return cc

            lax.fori_loop(0, nrows, mul_row, 0, unroll=4)

        def mul_scatter(b, wait_pred):
            if wait_pred is True:
                scatter_wait(b)
            elif wait_pred is not False:
                @pl.when(wait_pred)
                def _():
                    scatter_wait(b)
            offs = list(range(0, K - LANES + 1, LANES))
            if K % LANES:
                offs.append(K - LANES)
            for o in offs:
                sl = pl.ds(o, LANES)
                dst_sc[b][sl] = dst_v[b, sl]
            mul_rows(b, K)
            scatter_start(b)

        a_start(0, 0)
        a_wait(0, 0)
        g_start(0)
        a_start(1, 1)

        def pair_body(t, carry):
            g = 2 * t
            warm = t >= 1
            a_wait(g + 1, 1)
            g_start(1)
            g_wait(0)
            mul_scatter(0, warm)
            a_start(g + 2, 0)
            a_wait(g + 2, 0)
            g_start(0)
            g_wait(1)
            mul_scatter(1, warm)

            @pl.when(g + 3 < n_chunks)
            def _():
                a_start(g + 3, 1)

            return carry

        lax.fori_loop(0, (n_chunks - 1) // 2, pair_body, 0)
        if n_chunks % 2 == 1:
            g_wait(0)
            mul_scatter(0, True)
        else:
            a_wait(n_chunks - 1, 1)
            g_start(1)
            g_wait(0)
            mul_scatter(0, True)
            g_wait(1)
            mul_scatter(1, True)
        scatter_wait(0)
        scatter_wait(1)

        if rem:
            base = base0 + n_chunks * K
            pltpu.sync_copy(src_hbm.at[pl.ds(base, rem)], srcr_v)
            pltpu.sync_copy(dst_hbm.at[pl.ds(base, rem)], dstr_v)
            pltpu.sync_copy(r_hbm.at[pl.ds(base, rem)],
                            rv_v.at[0, pl.ds(0, rem)])
            pltpu.async_copy(xm_hbm.at[srcr_v], rows_v.at[0, pl.ds(0, rem)],
                             sem_g0).wait()
            mul_rows(0, rem)
            pltpu.sync_copy(msg_v.at[0, pl.ds(0, rem)], agg_s.at[dstr_v],
                            add=True)

        plsc.subcore_barrier()

        @pl.when(s < NS - 1)
        def _():
            pltpu.sync_copy(agg_s.at[pl.ds(s * NR, NR)],
                            out_hbm.at[pl.ds(c * N + s * NR, NR)])

        @pl.when(s == NS - 1)
        def _():
            pltpu.sync_copy(agg_s.at[pl.ds((NS - 1) * NR, NR_LAST)],
                            out_hbm.at[pl.ds(c * N + (NS - 1) * NR, NR_LAST)])

    return sc_scatter


def kernel(x, edge_index, edge_attr, W_msg, W_self, W_r, b_r, W_si, b_si,
           W_nl, b_nl, W_fin, b_fin):
    N, D = x.shape
    E = edge_index.shape[1]
    DE = edge_attr.shape[1]
    H = W_si.shape[1]
    C = W_fin.shape[1]

    BN = 1000
    xm = pl.pallas_call(
        _xm_body,
        grid=(N // BN,),
        in_specs=[
            pl.BlockSpec((BN, D), lambda i: (i, 0)),
            pl.BlockSpec((D, D), lambda i: (0, 0)),
        ],
        out_specs=pl.BlockSpec((BN, D), lambda i: (i, 0)),
        out_shape=jax.ShapeDtypeStruct((N, D), jnp.float32),
    )(x, W_msg)

    BE = 8000
    r = pl.pallas_call(
        _r_body,
        grid=(E // BE,),
        in_specs=[
            pl.BlockSpec((BE, DE), lambda i: (i, 0)),
            pl.BlockSpec((DE, D), lambda i: (0, 0)),
            pl.BlockSpec((1, D), lambda i: (0, 0)),
        ],
        out_specs=pl.BlockSpec((BE, D), lambda i: (i, 0)),
        out_shape=jax.ShapeDtypeStruct((E, D), jnp.float32),
    )(edge_attr, W_r, b_r.reshape(1, D))

    K = 64
    src = edge_index[0]
    dst = edge_index[1]
    zeros = jnp.zeros((N, D), jnp.float32)
    parts = _make_sc_scatter(N, E, D, K)(xm, r, src, dst, zeros)

    out = pl.pallas_call(
        _tail_body,
        grid=(N // BN,),
        in_specs=[
            pl.BlockSpec((BN, D), lambda i: (i, 0)),
            pl.BlockSpec((BN, D), lambda i: (i, 0)),
            pl.BlockSpec((BN, D), lambda i: (i + N // BN, 0)),
            pl.BlockSpec((D, D), lambda i: (0, 0)),
            pl.BlockSpec((D, H), lambda i: (0, 0)),
            pl.BlockSpec((1, H), lambda i: (0, 0)),
            pl.BlockSpec((H, H), lambda i: (0, 0)),
            pl.BlockSpec((1, H), lambda i: (0, 0)),
            pl.BlockSpec((H, C), lambda i: (0, 0)),
            pl.BlockSpec((1, C), lambda i: (0, 0)),
        ],
        out_specs=pl.BlockSpec((BN, C), lambda i: (i, 0)),
        out_shape=jax.ShapeDtypeStruct((N, C), jnp.float32),
    )(x, parts, parts, W_self, W_si, b_si.reshape(1, H), W_nl,
      b_nl.reshape(1, H), W_fin, b_fin.reshape(1, C))

    return out


# speedup vs baseline: 1.1357x; 1.0151x over previous
import functools

import jax
import jax.numpy as jnp
from jax import lax
from jax.experimental import pallas as pl
from jax.experimental.pallas import tpu as pltpu
from jax.experimental.pallas import tpu_sc as plsc

NC = 2
NS = 16
LANES = 16


def _xm_body(x_ref, w_ref, o_ref):
    o_ref[...] = jnp.dot(x_ref[...], w_ref[...], preferred_element_type=jnp.float32)


def _r_body(a_ref, w_ref, b_ref, o_ref):
    o_ref[...] = (
        jnp.dot(a_ref[...], w_ref[...], preferred_element_type=jnp.float32)
        + b_ref[...]
    )


def _tail_body(x_ref, a0_ref, a1_ref, wself_ref, wsi_ref, bsi_ref, wnl_ref,
               bnl_ref, wfin_ref, bfin_ref, o_ref):
    h = (
        jnp.dot(x_ref[...], wself_ref[...], preferred_element_type=jnp.float32)
        + a0_ref[...]
        + a1_ref[...]
    )
    h = jnp.dot(h, wsi_ref[...], preferred_element_type=jnp.float32) + bsi_ref[...]
    h = jnp.maximum(
        jnp.dot(h, wnl_ref[...], preferred_element_type=jnp.float32) + bnl_ref[...],
        0.0,
    )
    o_ref[...] = (
        jnp.dot(h, wfin_ref[...], preferred_element_type=jnp.float32) + bfin_ref[...]
    )


def _make_sc_scatter(N, E, D, K):
    E_core = E // NC
    E_tile = E_core // NS
    n_chunks = E_tile // K
    rem = E_tile - n_chunks * K
    assert rem == 0 or (rem % 8 == 0 and rem <= 128)
    NR = -(-(N // NS) // 8) * 8
    NR_LAST = N - (NS - 1) * NR
    assert NR_LAST > 0 and NR_LAST % 8 == 0
    mesh = plsc.VectorSubcoreMesh(core_axis_name="c", subcore_axis_name="s")

    assert n_chunks >= 3

    @functools.partial(
        pl.kernel,
        mesh=mesh,
        out_type=jax.ShapeDtypeStruct((NC * N, D), jnp.float32),
        scratch_types=[
            pltpu.VMEM((2, K), jnp.int32),
            pltpu.VMEM((2, K), jnp.int32),
            pltpu.VMEM((K,), jnp.int32),
            pltpu.VMEM((K,), jnp.int32),
            pltpu.VMEM((max(rem, 8),), jnp.int32),
            pltpu.VMEM((max(rem, 8),), jnp.int32),
            pltpu.VMEM((2, K, D), jnp.float32),
            pltpu.VMEM((2, K, D), jnp.float32),
            pltpu.VMEM((2, K, D), jnp.float32),
            pltpu.VMEM_SHARED((N, D), jnp.float32),
            pltpu.SemaphoreType.DMA,
            pltpu.SemaphoreType.DMA,
            pltpu.SemaphoreType.DMA,
            pltpu.SemaphoreType.DMA,
            pltpu.SemaphoreType.DMA,
            pltpu.SemaphoreType.DMA,
        ],
    )
    def sc_scatter(xm_hbm, r_hbm, src_hbm, dst_hbm, zeros_hbm, out_hbm,
                   src_v, dst_v, dst_sc0, dst_sc1, srcr_v, dstr_v, rows_v,
                   rv_v, msg_v, agg_s, sem_in0, sem_in1, sem_g0, sem_g1,
                   sem_s0, sem_s1):
        dst_sc = (dst_sc0, dst_sc1)
        c = lax.axis_index("c")
        s = lax.axis_index("s")

        @pl.when(s < NS - 1)
        def _():
            pltpu.sync_copy(zeros_hbm.at[pl.ds(s * NR, NR)],
                            agg_s.at[pl.ds(s * NR, NR)])

        @pl.when(s == NS - 1)
        def _():
            pltpu.sync_copy(zeros_hbm.at[pl.ds((NS - 1) * NR, NR_LAST)],
                            agg_s.at[pl.ds((NS - 1) * NR, NR_LAST)])

        plsc.subcore_barrier()

        base0 = c * E_core + s * E_tile
        sem_in = (sem_in0, sem_in1)
        sem_g = (sem_g0, sem_g1)
        sem_s = (sem_s0, sem_s1)

        def a_start(i, b):
            base = base0 + i * K
            pltpu.async_copy(src_hbm.at[pl.ds(base, K)], src_v.at[b], sem_in[b])
            pltpu.async_copy(dst_hbm.at[pl.ds(base, K)], dst_v.at[b], sem_in[b])
            pltpu.async_copy(r_hbm.at[pl.ds(base, K)], rv_v.at[b], sem_in[b])

        def a_wait(i, b):
            base = base0 + i * K
            pltpu.make_async_copy(src_hbm.at[pl.ds(base, K)], src_v.at[b],
                                  sem_in[b]).wait()
            pltpu.make_async_copy(dst_hbm.at[pl.ds(base, K)], dst_v.at[b],
                                  sem_in[b]).wait()
            pltpu.make_async_copy(r_hbm.at[pl.ds(base, K)], rv_v.at[b],
                                  sem_in[b]).wait()

        def g_start(b):
            pltpu.async_copy(xm_hbm.at[src_v.at[b]], rows_v.at[b], sem_g[b])

        def g_wait(b):
            pltpu.make_async_copy(xm_hbm.at[src_v.at[b]], rows_v.at[b],
                                  sem_g[b]).wait()

        def scatter_start(b):
            pltpu.async_copy(msg_v.at[b], agg_s.at[dst_sc[b]], sem_s[b],
                             add=True)

        def scatter_wait(b):
            pltpu.make_async_copy(msg_v.at[b], agg_s.at[dst_sc[b]],
                                  sem_s[b]).wait()

        def mul_rows(b, nrows):
            def mul_row(k, cc):
                prods = []
                for j in range(D // LANES):
                    sl = pl.ds(j * LANES, LANES)
                    prods.append(rows_v[b, k, sl] * rv_v[b, k, sl])
                for j in range(D // LANES):
                    sl = pl.ds(j * LANES, LANES)
                    msg_v[b, k, sl] = prods[j]
                return cc

            lax.fori_loop(0, nrows, mul_row, 0, unroll=4)

        def mul_scatter(b, wait_pred):
            if wait_pred is True:
                scatter_wait(b)
            elif wait_pred is not False:
                @pl.when(wait_pred)
                def _():
                    scatter_wait(b)
            offs = list(range(0, K - LANES + 1, LANES))
            if K % LANES:
                offs.append(K - LANES)
            for o in offs:
                sl = pl.ds(o, LANES)
                dst_sc[b][sl] = dst_v[b, sl]
            mul_rows(b, K)
            scatter_start(b)

        a_start(0, 0)
        a_wait(0, 0)
        g_start(0)
        a_start(1, 1)

        def pair_body(t, carry):
            g = 2 * t
            warm = t >= 1
            a_wait(g + 1, 1)
            g_start(1)
            g_wait(0)
            mul_scatter(0, warm)
            a_start(g + 2, 0)
            a_wait(g + 2, 0)
            g_start(0)
            g_wait(1)
            mul_scatter(1, warm)

            @pl.when(g + 3 < n_chunks)
            def _():
                a_start(g + 3, 1)

            return carry

        lax.fori_loop(0, (n_chunks - 1) // 2, pair_body, 0)
        if n_chunks % 2 == 1:
            g_wait(0)
            mul_scatter(0, True)
        else:
            a_wait(n_chunks - 1, 1)
            g_start(1)
            g_wait(0)
            mul_scatter(0, True)
            g_wait(1)
            mul_scatter(1, True)
        scatter_wait(0)
        scatter_wait(1)

        if rem:
            base = base0 + n_chunks * K
            pltpu.sync_copy(src_hbm.at[pl.ds(base, rem)], srcr_v)
            pltpu.sync_copy(dst_hbm.at[pl.ds(base, rem)], dstr_v)
            pltpu.sync_copy(r_hbm.at[pl.ds(base, rem)],
                            rv_v.at[0, pl.ds(0, rem)])
            pltpu.async_copy(xm_hbm.at[srcr_v], rows_v.at[0, pl.ds(0, rem)],
                             sem_g0).wait()
            mul_rows(0, rem)
            pltpu.sync_copy(msg_v.at[0, pl.ds(0, rem)], agg_s.at[dstr_v],
                            add=True)

        plsc.subcore_barrier()

        @pl.when(s < NS - 1)
        def _():
            pltpu.sync_copy(agg_s.at[pl.ds(s * NR, NR)],
                            out_hbm.at[pl.ds(c * N + s * NR, NR)])

        @pl.when(s == NS - 1)
        def _():
            pltpu.sync_copy(agg_s.at[pl.ds((NS - 1) * NR, NR_LAST)],
                            out_hbm.at[pl.ds(c * N + (NS - 1) * NR, NR_LAST)])

    return sc_scatter


def kernel(x, edge_index, edge_attr, W_msg, W_self, W_r, b_r, W_si, b_si,
           W_nl, b_nl, W_fin, b_fin):
    N, D = x.shape
    E = edge_index.shape[1]
    DE = edge_attr.shape[1]
    H = W_si.shape[1]
    C = W_fin.shape[1]

    BN = 2000
    xm = pl.pallas_call(
        _xm_body,
        grid=(N // BN,),
        in_specs=[
            pl.BlockSpec((BN, D), lambda i: (i, 0)),
            pl.BlockSpec((D, D), lambda i: (0, 0)),
        ],
        out_specs=pl.BlockSpec((BN, D), lambda i: (i, 0)),
        out_shape=jax.ShapeDtypeStruct((N, D), jnp.float32),
    )(x, W_msg)

    BE = 16000
    r = pl.pallas_call(
        _r_body,
        grid=(E // BE,),
        in_specs=[
            pl.BlockSpec((BE, DE), lambda i: (i, 0)),
            pl.BlockSpec((DE, D), lambda i: (0, 0)),
            pl.BlockSpec((1, D), lambda i: (0, 0)),
        ],
        out_specs=pl.BlockSpec((BE, D), lambda i: (i, 0)),
        out_shape=jax.ShapeDtypeStruct((E, D), jnp.float32),
    )(edge_attr, W_r, b_r.reshape(1, D))

    K = 64
    src = edge_index[0]
    dst = edge_index[1]
    zeros = jnp.zeros((N, D), jnp.float32)
    parts = _make_sc_scatter(N, E, D, K)(xm, r, src, dst, zeros)

    out = pl.pallas_call(
        _tail_body,
        grid=(N // BN,),
        in_specs=[
            pl.BlockSpec((BN, D), lambda i: (i, 0)),
            pl.BlockSpec((BN, D), lambda i: (i, 0)),
            pl.BlockSpec((BN, D), lambda i: (i + N // BN, 0)),
            pl.BlockSpec((D, D), lambda i: (0, 0)),
            pl.BlockSpec((D, H), lambda i: (0, 0)),
            pl.BlockSpec((1, H), lambda i: (0, 0)),
            pl.BlockSpec((H, H), lambda i: (0, 0)),
            pl.BlockSpec((1, H), lambda i: (0, 0)),
            pl.BlockSpec((H, C), lambda i: (0, 0)),
            pl.BlockSpec((1, C), lambda i: (0, 0)),
        ],
        out_specs=pl.BlockSpec((BN, C), lambda i: (i, 0)),
        out_shape=jax.ShapeDtypeStruct((N, C), jnp.float32),
    )(x, parts, parts, W_self, W_si, b_si.reshape(1, H), W_nl,
      b_nl.reshape(1, H), W_fin, b_fin.reshape(1, C))

    return out
